# Initial kernel scaffold; baseline (speedup 1.0000x reference)
import functools

import jax
import jax.numpy as jnp
from jax import lax
from jax.experimental import pallas as pl
from jax.experimental.pallas import tpu as pltpu
from jax.experimental.pallas import tpu_sc as plsc

N = 10000
E = 320000
D = 128
ALPHA = 0.2

NC = 2
NS = 16
NW = NC * NS
NPAD = 10240
E_PER_TILE = E // NW
CHUNK = 80
NCHUNK = E_PER_TILE // CHUNK
ROWS_STRIPE = NPAD // NS


def _mm_body(x_ref, w_ref, al_ref, ar_ref, wh_ref, el_ref, er_ref):
    wh = jnp.dot(x_ref[...], w_ref[...], preferred_element_type=jnp.float32)
    wh_ref[...] = wh
    el_ref[...] = jnp.sum(wh * al_ref[...], axis=1, keepdims=True)
    er_ref[...] = jnp.sum(wh * ar_ref[...], axis=1, keepdims=True)


def _matmul_part(x, W, a_left, a_right):
    BLK = 400
    grid = N // BLK
    return pl.pallas_call(
        _mm_body,
        grid=(grid,),
        in_specs=[
            pl.BlockSpec((BLK, D), lambda i: (i, 0)),
            pl.BlockSpec((D, D), lambda i: (0, 0)),
            pl.BlockSpec((1, D), lambda i: (0, 0)),
            pl.BlockSpec((1, D), lambda i: (0, 0)),
        ],
        out_specs=[
            pl.BlockSpec((BLK, D), lambda i: (i, 0)),
            pl.BlockSpec((BLK, 1), lambda i: (i, 0)),
            pl.BlockSpec((BLK, 1), lambda i: (i, 0)),
        ],
        out_shape=[
            jax.ShapeDtypeStruct((N, D), jnp.float32),
            jax.ShapeDtypeStruct((N, 1), jnp.float32),
            jax.ShapeDtypeStruct((N, 1), jnp.float32),
        ],
    )(x, W, a_left.reshape(1, D), a_right.reshape(1, D))


def _sc_edge_kernel(src2d, dst2d, el, er, wh):
    mesh = plsc.VectorSubcoreMesh(core_axis_name="c", subcore_axis_name="s")

    @functools.partial(
        pl.kernel,
        out_type=[
            jax.ShapeDtypeStruct((NC, NPAD, D), jnp.float32),
            jax.ShapeDtypeStruct((NW, NPAD), jnp.float32),
        ],
        mesh=mesh,
        scratch_types=[
            pltpu.VMEM((NCHUNK, CHUNK), jnp.int32),
            pltpu.VMEM((NCHUNK, CHUNK), jnp.int32),
            pltpu.VMEM((N,), jnp.float32),
            pltpu.VMEM((N,), jnp.float32),
            pltpu.VMEM((NPAD,), jnp.float32),
            pltpu.VMEM((CHUNK,), jnp.float32),
            pltpu.VMEM((CHUNK, D), jnp.float32),
            pltpu.VMEM_SHARED((NPAD, D), jnp.float32),
            pltpu.SemaphoreType.DMA,
        ],
    )
    def k(src_hbm, dst_hbm, el_hbm, er_hbm, wh_hbm, s_out, sum_out,
          src_v, dst_v, el_v, er_v, esum_v, p_v, rows_v, s_sh, sem):
        cid = lax.axis_index("c")
        sid = lax.axis_index("s")
        wid = cid * NS + sid
        zeros16 = jnp.zeros((16,), jnp.float32)

        def zero_esum(i, _):
            esum_v[pl.ds(i * 16, 16)] = zeros16
            return 0
        lax.fori_loop(0, NPAD // 16, zero_esum, 0)

        def zero_rows(i, _):
            for j in range(D // 16):
                rows_v[i, pl.ds(j * 16, 16)] = zeros16
            return 0
        lax.fori_loop(0, CHUNK, zero_rows, 0)

        stripe0 = sid * ROWS_STRIPE
        for q in range(ROWS_STRIPE // CHUNK):
            pltpu.sync_copy(rows_v, s_sh.at[pl.ds(stripe0 + q * CHUNK, CHUNK)])
        plsc.subcore_barrier()

        pltpu.sync_copy(src_hbm.at[pl.ds(wid * NCHUNK, NCHUNK)], src_v)
        pltpu.sync_copy(dst_hbm.at[pl.ds(wid * NCHUNK, NCHUNK)], dst_v)
        pltpu.sync_copy(el_hbm, el_v)
        pltpu.sync_copy(er_hbm, er_v)

        def chunk_body(c, _):
            pltpu.async_copy(wh_hbm.at[dst_v.at[c]], rows_v, sem).wait()

            for k16 in range(CHUNK // 16):
                s16 = src_v[c, pl.ds(k16 * 16, 16)]
                d16 = dst_v[c, pl.ds(k16 * 16, 16)]
                ev = plsc.load_gather(el_v, [s16]) + plsc.load_gather(er_v, [d16])
                ev = jnp.where(ev > 0, ev, ALPHA * ev)
                pv = jnp.exp(ev)
                p_v[pl.ds(k16 * 16, 16)] = pv
                plsc.addupdate_scatter(esum_v, [s16], pv)

            def scale_row(r, _):
                pr = p_v[r]
                for j in range(D // 16):
                    rows_v[r, pl.ds(j * 16, 16)] = rows_v[r, pl.ds(j * 16, 16)] * pr
                return 0
            lax.fori_loop(0, CHUNK, scale_row, 0)

            pltpu.sync_copy(rows_v, s_sh.at[src_v.at[c]], add=True)
            return 0

        lax.fori_loop(0, NCHUNK, chunk_body, 0)
        plsc.subcore_barrier()

        pltpu.sync_copy(esum_v, sum_out.at[wid])
        for q in range(ROWS_STRIPE // CHUNK):
            base = stripe0 + q * CHUNK
            pltpu.sync_copy(s_sh.at[pl.ds(base, CHUNK)], rows_v)
            pltpu.sync_copy(rows_v, s_out.at[cid, pl.ds(base, CHUNK)])

    return k(src2d, dst2d, el, er, wh)


def _combine_body(s_ref, sum_ref, o_ref):
    s = s_ref[0] + s_ref[1]
    d = jnp.sum(sum_ref[...], axis=0) + 1e-9
    o_ref[...] = jnp.maximum(s * (1.0 / d)[:, None], 0.0)


def _combine(s_parts, sum_parts):
    BLK = 512
    grid = NPAD // BLK
    return pl.pallas_call(
        _combine_body,
        grid=(grid,),
        in_specs=[
            pl.BlockSpec((NC, BLK, D), lambda i: (0, i, 0)),
            pl.BlockSpec((NW, BLK), lambda i: (0, i)),
        ],
        out_specs=pl.BlockSpec((BLK, D), lambda i: (i, 0)),
        out_shape=jax.ShapeDtypeStruct((NPAD, D), jnp.float32),
    )(s_parts, sum_parts)


def kernel(x, edge_index, W, a_left, a_right):
    wh, el2, er2 = _matmul_part(x, W, a_left, a_right)
    el = el2.reshape(N)
    er = er2.reshape(N)
    src2d = edge_index[0].reshape(NW * NCHUNK, CHUNK)
    dst2d = edge_index[1].reshape(NW * NCHUNK, CHUNK)
    s_parts, sum_parts = _sc_edge_kernel(src2d, dst2d, el, er, wh)
    out = _combine(s_parts, sum_parts)
    return out[:N]


# speedup vs baseline: 23.4843x; 23.4843x over previous
import functools

import jax
import jax.numpy as jnp
from jax import lax
from jax.experimental import pallas as pl
from jax.experimental.pallas import tpu as pltpu
from jax.experimental.pallas import tpu_sc as plsc

N = 10000
E = 320000
D = 128
ALPHA = 0.2

NC = 2
NS = 16
NW = NC * NS
NPAD = 10240
E_PER_TILE = E // NW
CHUNK = 80
NCHUNK = E_PER_TILE // CHUNK
ROWS_STRIPE = NPAD // NS


def _mm_body(x_ref, w_ref, al_ref, ar_ref, wh_ref, el_ref, er_ref):
    wh = jnp.dot(x_ref[...], w_ref[...], preferred_element_type=jnp.float32)
    wh_ref[...] = wh
    el_ref[...] = jnp.sum(wh * al_ref[...], axis=1, keepdims=True)
    er_ref[...] = jnp.sum(wh * ar_ref[...], axis=1, keepdims=True)


def _matmul_part(x, W, a_left, a_right):
    BLK = 400
    grid = N // BLK
    return pl.pallas_call(
        _mm_body,
        grid=(grid,),
        in_specs=[
            pl.BlockSpec((BLK, D), lambda i: (i, 0)),
            pl.BlockSpec((D, D), lambda i: (0, 0)),
            pl.BlockSpec((1, D), lambda i: (0, 0)),
            pl.BlockSpec((1, D), lambda i: (0, 0)),
        ],
        out_specs=[
            pl.BlockSpec((BLK, D), lambda i: (i, 0)),
            pl.BlockSpec((BLK, 1), lambda i: (i, 0)),
            pl.BlockSpec((BLK, 1), lambda i: (i, 0)),
        ],
        out_shape=[
            jax.ShapeDtypeStruct((N, D), jnp.float32),
            jax.ShapeDtypeStruct((N, 1), jnp.float32),
            jax.ShapeDtypeStruct((N, 1), jnp.float32),
        ],
    )(x, W, a_left.reshape(1, D), a_right.reshape(1, D))


def _sc_edge_kernel(src2d, dst2d, el, er, wh):
    mesh = plsc.VectorSubcoreMesh(core_axis_name="c", subcore_axis_name="s")

    @functools.partial(
        pl.kernel,
        out_type=[
            jax.ShapeDtypeStruct((NC, NPAD, D), jnp.float32),
            jax.ShapeDtypeStruct((NW, NPAD), jnp.float32),
        ],
        mesh=mesh,
        compiler_params=pltpu.CompilerParams(needs_layout_passes=False),
        scratch_types=[
            pltpu.VMEM((CHUNK,), jnp.int32),
            pltpu.VMEM((CHUNK,), jnp.int32),
            pltpu.VMEM((N,), jnp.float32),
            pltpu.VMEM((N,), jnp.float32),
            pltpu.VMEM((NPAD,), jnp.float32),
            pltpu.VMEM((CHUNK, D), jnp.float32),
            pltpu.VMEM_SHARED((NPAD, D), jnp.float32),
            pltpu.SemaphoreType.DMA,
        ],
    )
    def k(src_hbm, dst_hbm, el_hbm, er_hbm, wh_hbm, s_out, sum_out,
          src_v, dst_v, el_v, er_v, esum_v, rows_v, s_sh, sem):
        cid = lax.axis_index("c")
        sid = lax.axis_index("s")
        wid = cid * NS + sid
        zeros16 = jnp.zeros((16,), jnp.float32)

        def zero_esum(i, _):
            esum_v[pl.ds(i * 16, 16)] = zeros16
            return 0
        lax.fori_loop(0, NPAD // 16, zero_esum, 0)

        def zero_rows(i, _):
            for j in range(D // 16):
                rows_v[i, pl.ds(j * 16, 16)] = zeros16
            return 0
        lax.fori_loop(0, CHUNK, zero_rows, 0)

        stripe0 = sid * ROWS_STRIPE
        for q in range(ROWS_STRIPE // CHUNK):
            pltpu.sync_copy(rows_v, s_sh.at[pl.ds(stripe0 + q * CHUNK, CHUNK)])
        plsc.subcore_barrier()

        pltpu.sync_copy(el_hbm, el_v)
        pltpu.sync_copy(er_hbm, er_v)

        def chunk_body(c, _):
            pltpu.sync_copy(src_hbm.at[wid, c], src_v)
            pltpu.sync_copy(dst_hbm.at[wid, c], dst_v)
            pltpu.async_copy(wh_hbm.at[dst_v], rows_v, sem).wait()

            for k16 in range(CHUNK // 16):
                s16 = src_v[pl.ds(k16 * 16, 16)]
                d16 = dst_v[pl.ds(k16 * 16, 16)]
                ev = plsc.load_gather(el_v, [s16]) + plsc.load_gather(er_v, [d16])
                ev = jnp.where(ev > 0, ev, ALPHA * ev)
                pv = jnp.exp(ev)
                plsc.addupdate_scatter(esum_v, [s16], pv)
                for i in range(16):
                    r = k16 * 16 + i
                    pr = pv[i]
                    for j in range(D // 16):
                        rows_v[r, pl.ds(j * 16, 16)] = (
                            rows_v[r, pl.ds(j * 16, 16)] * pr)

            pltpu.sync_copy(rows_v, s_sh.at[src_v], add=True)
            return 0

        lax.fori_loop(0, NCHUNK, chunk_body, 0)
        plsc.subcore_barrier()

        pltpu.sync_copy(esum_v, sum_out.at[wid])
        for q in range(ROWS_STRIPE // CHUNK):
            base = stripe0 + q * CHUNK
            pltpu.sync_copy(s_sh.at[pl.ds(base, CHUNK)], rows_v)
            pltpu.sync_copy(rows_v, s_out.at[cid, pl.ds(base, CHUNK)])

    return k(src2d, dst2d, el, er, wh)


def _combine_body(s_ref, sum_ref, o_ref):
    s = s_ref[0] + s_ref[1]
    d = jnp.sum(sum_ref[...], axis=0) + 1e-9
    o_ref[...] = jnp.maximum(s * (1.0 / d)[:, None], 0.0)


def _combine(s_parts, sum_parts):
    BLK = 512
    grid = NPAD // BLK
    return pl.pallas_call(
        _combine_body,
        grid=(grid,),
        in_specs=[
            pl.BlockSpec((NC, BLK, D), lambda i: (0, i, 0)),
            pl.BlockSpec((NW, BLK), lambda i: (0, i)),
        ],
        out_specs=pl.BlockSpec((BLK, D), lambda i: (i, 0)),
        out_shape=jax.ShapeDtypeStruct((NPAD, D), jnp.float32),
    )(s_parts, sum_parts)


def kernel(x, edge_index, W, a_left, a_right):
    wh, el2, er2 = _matmul_part(x, W, a_left, a_right)
    el = el2.reshape(N)
    er = er2.reshape(N)
    src2d = edge_index[0].reshape(NW, NCHUNK, CHUNK)
    dst2d = edge_index[1].reshape(NW, NCHUNK, CHUNK)
    s_parts, sum_parts = _sc_edge_kernel(src2d, dst2d, el, er, wh)
    out = _combine(s_parts, sum_parts)
    return out[:N]


# speedup vs baseline: 30.5472x; 1.3008x over previous
import functools

import jax
import jax.numpy as jnp
from jax import lax
from jax.experimental import pallas as pl
from jax.experimental.pallas import tpu as pltpu
from jax.experimental.pallas import tpu_sc as plsc

N = 10000
E = 320000
D = 128
ALPHA = 0.2

NC = 2
NS = 16
NW = NC * NS
NPAD = 10240
CHUNK = 80
NCHUNK = 126
E_PER_TILE = NCHUNK * CHUNK
E_PAD = NW * E_PER_TILE
ROWS_STRIPE = NPAD // NS
NBUF = 2


def _mm_body(x_ref, w_ref, al_ref, ar_ref, wh_ref, el_ref, er_ref):
    wh = jnp.dot(x_ref[...], w_ref[...], preferred_element_type=jnp.float32)
    wh_ref[...] = wh
    el_ref[...] = jnp.sum(wh * al_ref[...], axis=1, keepdims=True)
    er_ref[...] = jnp.sum(wh * ar_ref[...], axis=1, keepdims=True)


def _matmul_part(x, W, a_left, a_right):
    BLK = 400
    grid = N // BLK
    return pl.pallas_call(
        _mm_body,
        grid=(grid,),
        in_specs=[
            pl.BlockSpec((BLK, D), lambda i: (i, 0)),
            pl.BlockSpec((D, D), lambda i: (0, 0)),
            pl.BlockSpec((1, D), lambda i: (0, 0)),
            pl.BlockSpec((1, D), lambda i: (0, 0)),
        ],
        out_specs=[
            pl.BlockSpec((BLK, D), lambda i: (i, 0)),
            pl.BlockSpec((BLK, 1), lambda i: (i, 0)),
            pl.BlockSpec((BLK, 1), lambda i: (i, 0)),
        ],
        out_shape=[
            jax.ShapeDtypeStruct((N, D), jnp.float32),
            jax.ShapeDtypeStruct((N, 1), jnp.float32),
            jax.ShapeDtypeStruct((N, 1), jnp.float32),
        ],
    )(x, W, a_left.reshape(1, D), a_right.reshape(1, D))


def _sc_edge_kernel(src3d, dst3d, el, er, wh):
    mesh = plsc.VectorSubcoreMesh(core_axis_name="c", subcore_axis_name="s")

    @functools.partial(
        pl.kernel,
        out_type=[
            jax.ShapeDtypeStruct((NC, NPAD, D), jnp.float32),
            jax.ShapeDtypeStruct((NC, NPAD), jnp.float32),
        ],
        mesh=mesh,
        compiler_params=pltpu.CompilerParams(needs_layout_passes=False),
        scratch_types=[
            pltpu.VMEM((CHUNK, D), jnp.float32),
            pltpu.VMEM((CHUNK, D), jnp.float32),
            pltpu.VMEM((CHUNK,), jnp.int32),
            pltpu.VMEM((CHUNK,), jnp.int32),
            pltpu.VMEM((CHUNK,), jnp.int32),
            pltpu.VMEM((CHUNK,), jnp.int32),
            pltpu.VMEM((CHUNK,), jnp.int32),
            pltpu.VMEM((CHUNK,), jnp.int32),
            pltpu.VMEM((CHUNK,), jnp.float32),
            pltpu.VMEM((CHUNK,), jnp.float32),
            pltpu.VMEM((CHUNK,), jnp.float32),
            pltpu.VMEM((CHUNK,), jnp.float32),
            pltpu.VMEM((CHUNK,), jnp.float32),
            pltpu.VMEM((CHUNK,), jnp.float32),
            pltpu.VMEM((ROWS_STRIPE,), jnp.float32),
            pltpu.VMEM_SHARED((NPAD, D), jnp.float32),
            pltpu.VMEM_SHARED((NPAD,), jnp.float32),
            pltpu.SemaphoreType.DMA,
            pltpu.SemaphoreType.DMA,
            pltpu.SemaphoreType.DMA,
            pltpu.SemaphoreType.DMA,
        ],
    )
    def k(src_hbm, dst_hbm, el_hbm, er_hbm, wh_hbm, s_out, sum_out,
          rows0, rows1, si0, si1, di0, di1, ci0, ci1,
          el0, el1, er0, er1, pv0, pv1,
          zb, s_sh, esum_sh, sem0, sem1, isem0, isem1):
        cid = lax.axis_index("c")
        sid = lax.axis_index("s")
        wid = cid * NS + sid
        zeros16 = jnp.zeros((16,), jnp.float32)
        bufs = (
            (rows0, si0, di0, ci0, el0, er0, pv0, sem0, isem0),
            (rows1, si1, di1, ci1, el1, er1, pv1, sem1, isem1),
        )

        def zero_rows(i, _):
            for j in range(D // 16):
                rows0[i, pl.ds(j * 16, 16)] = zeros16
            return 0
        lax.fori_loop(0, CHUNK, zero_rows, 0)

        def zero_zb(i, _):
            zb[pl.ds(i * 16, 16)] = zeros16
            return 0
        lax.fori_loop(0, ROWS_STRIPE // 16, zero_zb, 0)

        stripe0 = sid * ROWS_STRIPE
        for q in range(ROWS_STRIPE // CHUNK):
            pltpu.sync_copy(rows0, s_sh.at[pl.ds(stripe0 + q * CHUNK, CHUNK)])
        pltpu.sync_copy(zb, esum_sh.at[pl.ds(stripe0, ROWS_STRIPE)])
        plsc.subcore_barrier()

        def fetch_idx(c, b):
            _, si_b, di_b, _, _, _, _, _, isem_b = bufs[b]
            pltpu.async_copy(src_hbm.at[wid, c], si_b, isem_b)
            pltpu.async_copy(dst_hbm.at[wid, c], di_b, isem_b)

        def drain_idx(b):
            _, si_b, di_b, _, _, _, _, _, isem_b = bufs[b]
            pltpu.make_async_copy(src_hbm.at[wid, 0], si_b, isem_b).wait()
            pltpu.make_async_copy(src_hbm.at[wid, 0], di_b, isem_b).wait()

        def issue_gathers(b):
            rows_b, si_b, di_b, _, el_b, er_b, _, sem_b, _ = bufs[b]
            pltpu.async_copy(wh_hbm.at[di_b], rows_b, sem_b)
            pltpu.async_copy(el_hbm.at[si_b], el_b, sem_b)
            pltpu.async_copy(er_hbm.at[di_b], er_b, sem_b)

        fetch_idx(0, 0)
        fetch_idx(1, 1)
        drain_idx(0)
        issue_gathers(0)
        drain_idx(1)
        issue_gathers(1)

        @pl.loop(0, NCHUNK, step=2)
        def chunk_pair(g):
            for b in range(2):
                c = g + b
                rows_b, si_b, di_b, ci_b, el_b, er_b, pv_b, sem_b, _ = bufs[b]
                pltpu.make_async_copy(
                    wh_hbm.at[pl.ds(0, CHUNK)], rows_b, sem_b).wait()
                pltpu.make_async_copy(
                    el_hbm.at[pl.ds(0, CHUNK)], el_b, sem_b).wait()
                pltpu.make_async_copy(
                    el_hbm.at[pl.ds(0, CHUNK)], er_b, sem_b).wait()

                for k16 in range(CHUNK // 16):
                    ci_b[pl.ds(k16 * 16, 16)] = si_b[pl.ds(k16 * 16, 16)]

                @pl.when(c + 2 < NCHUNK)
                def _():
                    fetch_idx(c + 2, b)

                for gi in range(CHUNK // 16):
                    ev = el_b[pl.ds(gi * 16, 16)] + er_b[pl.ds(gi * 16, 16)]
                    ev = jnp.where(ev > 0, ev, ALPHA * ev)
                    pvv = jnp.exp(ev)
                    pv_b[pl.ds(gi * 16, 16)] = pvv
                    for i in range(16):
                        r = gi * 16 + i
                        pr = pvv[i]
                        for j in range(D // 16):
                            rows_b[r, pl.ds(j * 16, 16)] = (
                                rows_b[r, pl.ds(j * 16, 16)] * pr)

                pltpu.sync_copy(pv_b, esum_sh.at[ci_b], add=True)
                pltpu.sync_copy(rows_b, s_sh.at[ci_b], add=True)

                @pl.when(c + 2 < NCHUNK)
                def _():
                    drain_idx(b)
                    issue_gathers(b)

        plsc.subcore_barrier()

        pltpu.sync_copy(esum_sh.at[pl.ds(stripe0, ROWS_STRIPE)], zb)
        pltpu.sync_copy(zb, sum_out.at[cid, pl.ds(stripe0, ROWS_STRIPE)])
        for q in range(ROWS_STRIPE // CHUNK):
            base = stripe0 + q * CHUNK
            pltpu.sync_copy(s_sh.at[pl.ds(base, CHUNK)], rows0)
            pltpu.sync_copy(rows0, s_out.at[cid, pl.ds(base, CHUNK)])

    return k(src3d, dst3d, el, er, wh)


def _combine_body(s_ref, sum_ref, o_ref):
    s = s_ref[0] + s_ref[1]
    d = sum_ref[0] + sum_ref[1] + 1e-9
    o_ref[...] = jnp.maximum(s * (1.0 / d)[:, None], 0.0)


def _combine(s_parts, sum_parts):
    BLK = 512
    grid = NPAD // BLK
    return pl.pallas_call(
        _combine_body,
        grid=(grid,),
        in_specs=[
            pl.BlockSpec((NC, BLK, D), lambda i: (0, i, 0)),
            pl.BlockSpec((NC, BLK), lambda i: (0, i)),
        ],
        out_specs=pl.BlockSpec((BLK, D), lambda i: (i, 0)),
        out_shape=jax.ShapeDtypeStruct((NPAD, D), jnp.float32),
    )(s_parts, sum_parts)


def kernel(x, edge_index, W, a_left, a_right):
    wh, el2, er2 = _matmul_part(x, W, a_left, a_right)
    el = jnp.pad(el2.reshape(N), (0, NPAD - N))
    er = jnp.pad(er2.reshape(N), (0, NPAD - N))
    src = jnp.concatenate(
        [edge_index[0], jnp.full((E_PAD - E,), N, jnp.int32)])
    dst = jnp.concatenate(
        [edge_index[1], jnp.zeros((E_PAD - E,), jnp.int32)])
    src3d = src.reshape(NW, NCHUNK, CHUNK)
    dst3d = dst.reshape(NW, NCHUNK, CHUNK)
    s_parts, sum_parts = _sc_edge_kernel(src3d, dst3d, el, er, wh)
    out = _combine(s_parts, sum_parts)
    return out[:N]
